# baseline (speedup 1.0000x reference)
import functools

import jax
import jax.numpy as jnp
from jax import lax
from jax.experimental import pallas as pl
from jax.experimental.pallas import tpu as pltpu
from jax.experimental.pallas import tpu_sc as plsc

N = 10000
E = 320000
D = 128
HOPS = 3
SIGMA = 0.1

NC = 2
NS = 16
NW = NC * NS
EPW = E // NW
KB = 80
NB = EPW // KB
RPT = 624
RTAIL = N - NS * RPT
ZR = KB
NZ = RPT // ZR
ZREM = RPT - NZ * ZR

_SELU_SCALE = 1.0507009873554805
_SELU_ALPHA = 1.6732632423543772


def _sc_hop_body(h_hbm, s12_hbm, src_hbm, dst_hbm, bk_hbm, out_hbm,
                 accum, s12v, rows0, rows1, srcb0, dstb0, srcb1, dstb1,
                 zbuf, bv, semi0, semi1, semr0, semr1):
    c = lax.axis_index("c")
    s = lax.axis_index("s")
    wid = c * NS + s
    ebase = wid * EPW
    rbase = s * RPT

    zero16 = jnp.zeros((16,), jnp.float32)

    def zrow(j, carry):
        for q in range(D // 16):
            zbuf[j, pl.ds(q * 16, 16)] = zero16
        return carry

    lax.fori_loop(0, ZR, zrow, 0)
    for z in range(NZ):
        pltpu.sync_copy(zbuf, accum.at[pl.ds(rbase + z * ZR, ZR)])
    pltpu.sync_copy(zbuf.at[pl.ds(0, ZREM)],
                    accum.at[pl.ds(rbase + NZ * ZR, ZREM)])

    @pl.when(s == 0)
    def _zero_tail():
        pltpu.sync_copy(zbuf.at[pl.ds(0, RTAIL)],
                        accum.at[pl.ds(NS * RPT, RTAIL)])

    plsc.subcore_barrier()

    pltpu.sync_copy(s12_hbm, s12v)
    pltpu.sync_copy(bk_hbm, bv)
    bvec = bv[...]

    bufs = ((srcb0, dstb0, rows0, semi0, semr0),
            (srcb1, dstb1, rows1, semi1, semr1))

    def idx_start(t, b):
        sb, db, _, semi, _ = bufs[b]
        eb = ebase + t * KB
        pltpu.make_async_copy(src_hbm.at[pl.ds(eb, KB)], sb, semi).start()
        pltpu.make_async_copy(dst_hbm.at[pl.ds(eb, KB)], db, semi).start()

    def idx_wait(b):
        sb, db, _, semi, _ = bufs[b]
        pltpu.make_async_copy(src_hbm.at[pl.ds(0, KB)], sb, semi).wait()
        pltpu.make_async_copy(dst_hbm.at[pl.ds(0, KB)], db, semi).wait()

    def gather_start(b):
        sb, _, rw, _, semr = bufs[b]
        pltpu.make_async_copy(h_hbm.at[sb], rw, semr).start()

    def gather_wait(b):
        sb, _, rw, _, semr = bufs[b]
        pltpu.make_async_copy(h_hbm.at[sb], rw, semr).wait()

    def compute_scatter(b):
        sb, db, rw, _, _ = bufs[b]
        for q in range(KB // 16):
            si = sb[pl.ds(q * 16, 16)]
            di = db[pl.ds(q * 16, 16)]
            ws = plsc.load_gather(s12v, [si])
            wd = plsc.load_gather(s12v, [di])
            s1f = plsc.bitcast(ws << 16, jnp.float32)
            s2f = plsc.bitcast(wd & jnp.int32(-65536), jnp.float32)
            e = s1f + s2f + bvec
            selu = _SELU_SCALE * jnp.where(
                e > 0, e, _SELU_ALPHA * (jnp.exp(e) - 1.0))
            avec = 1.0 / (1.0 + jnp.exp(-selu))
            for l in range(16):
                j = q * 16 + l
                av = avec[l]
                for qq in range(D // 16):
                    zbuf[j, pl.ds(qq * 16, 16)] = (
                        rw[j, pl.ds(qq * 16, 16)] * av)
        pltpu.sync_copy(zbuf, accum.at[db], add=True)

    idx_start(0, 0)
    idx_start(1, 1)
    idx_wait(0)
    gather_start(0)

    def bbody(i, carry):
        gather_wait(0)
        idx_wait(1)
        gather_start(1)
        compute_scatter(0)
        idx_start(2 * i + 2, 0)
        gather_wait(1)
        idx_wait(0)
        gather_start(0)
        compute_scatter(1)

        @pl.when(i < (NB - 1) // 2 - 1)
        def _more():
            idx_start(2 * i + 3, 1)

        return carry

    lax.fori_loop(0, (NB - 1) // 2, bbody, 0)
    gather_wait(0)
    compute_scatter(0)
    plsc.subcore_barrier()

    pltpu.sync_copy(accum.at[pl.ds(rbase, RPT)],
                    out_hbm.at[c, pl.ds(rbase, RPT)])

    @pl.when(s == 0)
    def _copy_tail():
        pltpu.sync_copy(accum.at[pl.ds(NS * RPT, RTAIL)],
                        out_hbm.at[c, pl.ds(NS * RPT, RTAIL)])


_sc_hop = functools.partial(
    pl.kernel,
    out_type=jax.ShapeDtypeStruct((NC, N, D), jnp.float32),
    mesh=plsc.VectorSubcoreMesh(core_axis_name="c", subcore_axis_name="s"),
    compiler_params=pltpu.CompilerParams(needs_layout_passes=False),
    scratch_types=[
        pltpu.VMEM_SHARED((N, D), jnp.float32),
        pltpu.VMEM((N,), jnp.int32),
        pltpu.VMEM((KB, D), jnp.float32),
        pltpu.VMEM((KB, D), jnp.float32),
        pltpu.VMEM((KB,), jnp.int32),
        pltpu.VMEM((KB,), jnp.int32),
        pltpu.VMEM((KB,), jnp.int32),
        pltpu.VMEM((KB,), jnp.int32),
        pltpu.VMEM((ZR, D), jnp.float32),
        pltpu.VMEM((16,), jnp.float32),
        pltpu.SemaphoreType.DMA,
        pltpu.SemaphoreType.DMA,
        pltpu.SemaphoreType.DMA,
        pltpu.SemaphoreType.DMA,
    ],
)(_sc_hop_body)


def _prep_body(x_ref, wp_ref, h_ref, sp_ref):
    xx = x_ref[...]
    nrm = jnp.sqrt(jnp.sum(xx * xx, axis=1, keepdims=True))
    h = xx / jnp.maximum(nrm, 1e-12)
    h_ref[...] = h
    sp_ref[...] = jnp.dot(h, wp_ref[...], preferred_element_type=jnp.float32)


def _tc_prep(x, wpad):
    return pl.pallas_call(
        _prep_body,
        out_shape=[jax.ShapeDtypeStruct((N, D), jnp.float32),
                   jax.ShapeDtypeStruct((N, 128), jnp.float32)],
    )(x, wpad)


def _combine_body(p_ref, nz_ref, wp_ref, h_ref, sp_ref):
    y = p_ref[0] + p_ref[1] + nz_ref[...]
    nrm = jnp.sqrt(jnp.sum(y * y, axis=1, keepdims=True))
    h = y / jnp.maximum(nrm, 1e-12)
    h_ref[...] = h
    sp_ref[...] = jnp.dot(h, wp_ref[...], preferred_element_type=jnp.float32)


def _tc_combine(partial, noise, wpad):
    return pl.pallas_call(
        _combine_body,
        out_shape=[jax.ShapeDtypeStruct((N, D), jnp.float32),
                   jax.ShapeDtypeStruct((N, 128), jnp.float32)],
    )(partial, noise, wpad)


def kernel(x, edge_index, W, b):
    src = edge_index[0]
    dst = edge_index[1]
    wpads = []
    for k in range(HOPS):
        wp = jnp.zeros((D, 128), jnp.float32)
        wp = wp.at[:, 0].set(W[k, :D]).at[:, 1].set(W[k, D:])
        wpads.append(wp)
    wpad_zero = jnp.zeros((D, 128), jnp.float32)

    h, sp = _tc_prep(x, wpads[0])
    outs = [h]
    for k in range(HOPS):
        b1 = jax.lax.bitcast_convert_type(
            sp[:, 0].astype(jnp.bfloat16), jnp.uint16).astype(jnp.uint32)
        b2 = jax.lax.bitcast_convert_type(
            sp[:, 1].astype(jnp.bfloat16), jnp.uint16).astype(jnp.uint32)
        s12 = jax.lax.bitcast_convert_type(b1 | (b2 << 16), jnp.int32)
        bk = jnp.full((16,), b[k], jnp.float32)
        partial = _sc_hop(h, s12, src, dst, bk)
        noise = SIGMA * jax.random.normal(
            jax.random.fold_in(jax.random.key(1), k), (N, D),
            dtype=jnp.float32)
        wnext = wpads[k + 1] if k + 1 < HOPS else wpad_zero
        h, sp = _tc_combine(partial, noise, wnext)
        outs.append(h)
    return jnp.stack(outs)


# speedup vs baseline: 1.0022x; 1.0022x over previous
import functools

import jax
import jax.numpy as jnp
from jax import lax
from jax.experimental import pallas as pl
from jax.experimental.pallas import tpu as pltpu
from jax.experimental.pallas import tpu_sc as plsc

N = 10000
E = 320000
D = 128
HOPS = 3
SIGMA = 0.1

NC = 2
NS = 16
NW = NC * NS
EPW = E // NW
KB = 80
NB = EPW // KB
RPT = 624
RTAIL = N - NS * RPT
ZR = KB
NZ = RPT // ZR
ZREM = RPT - NZ * ZR

_SELU_SCALE = 1.0507009873554805
_SELU_ALPHA = 1.6732632423543772


def _sc_hop_body(h_hbm, s12_hbm, src_hbm, dst_hbm, bk_hbm, out_hbm,
                 accum, s12v, rows0, rows1, srcb0, dstb0, srcb1, dstb1,
                 zbuf, bv, semi0, semi1, semr0, semr1):
    c = lax.axis_index("c")
    s = lax.axis_index("s")
    wid = c * NS + s
    ebase = wid * EPW
    rbase = s * RPT

    zero16 = jnp.zeros((16,), jnp.float32)

    def zrow(j, carry):
        for q in range(D // 16):
            zbuf[j, pl.ds(q * 16, 16)] = zero16
        return carry

    lax.fori_loop(0, ZR, zrow, 0)
    for z in range(NZ):
        pltpu.sync_copy(zbuf, accum.at[pl.ds(rbase + z * ZR, ZR)])
    pltpu.sync_copy(zbuf.at[pl.ds(0, ZREM)],
                    accum.at[pl.ds(rbase + NZ * ZR, ZREM)])

    @pl.when(s == 0)
    def _zero_tail():
        pltpu.sync_copy(zbuf.at[pl.ds(0, RTAIL)],
                        accum.at[pl.ds(NS * RPT, RTAIL)])

    plsc.subcore_barrier()

    pltpu.sync_copy(s12_hbm, s12v)
    pltpu.sync_copy(bk_hbm, bv)
    bvec = bv[...]

    bufs = ((srcb0, dstb0, rows0, semi0, semr0),
            (srcb1, dstb1, rows1, semi1, semr1))

    def idx_start(t, b):
        sb, db, _, semi, _ = bufs[b]
        eb = ebase + t * KB
        pltpu.make_async_copy(src_hbm.at[pl.ds(eb, KB)], sb, semi).start()
        pltpu.make_async_copy(dst_hbm.at[pl.ds(eb, KB)], db, semi).start()

    def idx_wait(b):
        sb, db, _, semi, _ = bufs[b]
        pltpu.make_async_copy(src_hbm.at[pl.ds(0, KB)], sb, semi).wait()
        pltpu.make_async_copy(dst_hbm.at[pl.ds(0, KB)], db, semi).wait()

    def gather_start(b):
        sb, _, rw, _, semr = bufs[b]
        pltpu.make_async_copy(h_hbm.at[sb], rw, semr).start()

    def gather_wait(b):
        sb, _, rw, _, semr = bufs[b]
        pltpu.make_async_copy(h_hbm.at[sb], rw, semr).wait()

    def compute_scatter(b):
        sb, db, rw, _, _ = bufs[b]
        for q in range(KB // 16):
            si = sb[pl.ds(q * 16, 16)]
            di = db[pl.ds(q * 16, 16)]
            ws = plsc.load_gather(s12v, [si])
            wd = plsc.load_gather(s12v, [di])
            s1f = plsc.bitcast(ws << 16, jnp.float32)
            s2f = plsc.bitcast(wd & jnp.int32(-65536), jnp.float32)
            e = s1f + s2f + bvec
            selu = _SELU_SCALE * jnp.where(
                e > 0, e, _SELU_ALPHA * (jnp.exp(e) - 1.0))
            avec = 1.0 / (1.0 + jnp.exp(-selu))
            for l in range(16):
                j = q * 16 + l
                av = avec[l]
                for qq in range(D // 16):
                    rw[j, pl.ds(qq * 16, 16)] = (
                        rw[j, pl.ds(qq * 16, 16)] * av)
        pltpu.sync_copy(rw, accum.at[db], add=True)

    idx_start(0, 0)
    idx_start(1, 1)
    idx_wait(0)
    gather_start(0)

    def bbody(i, carry):
        gather_wait(0)
        idx_wait(1)
        gather_start(1)
        compute_scatter(0)
        idx_start(2 * i + 2, 0)
        gather_wait(1)
        idx_wait(0)
        gather_start(0)
        compute_scatter(1)

        @pl.when(i < (NB - 1) // 2 - 1)
        def _more():
            idx_start(2 * i + 3, 1)

        return carry

    lax.fori_loop(0, (NB - 1) // 2, bbody, 0)
    gather_wait(0)
    compute_scatter(0)
    plsc.subcore_barrier()

    pltpu.sync_copy(accum.at[pl.ds(rbase, RPT)],
                    out_hbm.at[c, pl.ds(rbase, RPT)])

    @pl.when(s == 0)
    def _copy_tail():
        pltpu.sync_copy(accum.at[pl.ds(NS * RPT, RTAIL)],
                        out_hbm.at[c, pl.ds(NS * RPT, RTAIL)])


_sc_hop = functools.partial(
    pl.kernel,
    out_type=jax.ShapeDtypeStruct((NC, N, D), jnp.float32),
    mesh=plsc.VectorSubcoreMesh(core_axis_name="c", subcore_axis_name="s"),
    compiler_params=pltpu.CompilerParams(needs_layout_passes=False),
    scratch_types=[
        pltpu.VMEM_SHARED((N, D), jnp.float32),
        pltpu.VMEM((N,), jnp.int32),
        pltpu.VMEM((KB, D), jnp.float32),
        pltpu.VMEM((KB, D), jnp.float32),
        pltpu.VMEM((KB,), jnp.int32),
        pltpu.VMEM((KB,), jnp.int32),
        pltpu.VMEM((KB,), jnp.int32),
        pltpu.VMEM((KB,), jnp.int32),
        pltpu.VMEM((ZR, D), jnp.float32),
        pltpu.VMEM((16,), jnp.float32),
        pltpu.SemaphoreType.DMA,
        pltpu.SemaphoreType.DMA,
        pltpu.SemaphoreType.DMA,
        pltpu.SemaphoreType.DMA,
    ],
)(_sc_hop_body)


def _prep_body(x_ref, wp_ref, h_ref, sp_ref):
    xx = x_ref[...]
    nrm = jnp.sqrt(jnp.sum(xx * xx, axis=1, keepdims=True))
    h = xx / jnp.maximum(nrm, 1e-12)
    h_ref[...] = h
    sp_ref[...] = jnp.dot(h, wp_ref[...], preferred_element_type=jnp.float32)


def _tc_prep(x, wpad):
    return pl.pallas_call(
        _prep_body,
        out_shape=[jax.ShapeDtypeStruct((N, D), jnp.float32),
                   jax.ShapeDtypeStruct((N, 128), jnp.float32)],
    )(x, wpad)


def _combine_body(p_ref, nz_ref, wp_ref, h_ref, sp_ref):
    y = p_ref[0] + p_ref[1] + nz_ref[...]
    nrm = jnp.sqrt(jnp.sum(y * y, axis=1, keepdims=True))
    h = y / jnp.maximum(nrm, 1e-12)
    h_ref[...] = h
    sp_ref[...] = jnp.dot(h, wp_ref[...], preferred_element_type=jnp.float32)


def _tc_combine(partial, noise, wpad):
    return pl.pallas_call(
        _combine_body,
        out_shape=[jax.ShapeDtypeStruct((N, D), jnp.float32),
                   jax.ShapeDtypeStruct((N, 128), jnp.float32)],
    )(partial, noise, wpad)


def kernel(x, edge_index, W, b):
    src = edge_index[0]
    dst = edge_index[1]
    wpads = []
    for k in range(HOPS):
        wp = jnp.zeros((D, 128), jnp.float32)
        wp = wp.at[:, 0].set(W[k, :D]).at[:, 1].set(W[k, D:])
        wpads.append(wp)
    wpad_zero = jnp.zeros((D, 128), jnp.float32)

    h, sp = _tc_prep(x, wpads[0])
    outs = [h]
    for k in range(HOPS):
        b1 = jax.lax.bitcast_convert_type(
            sp[:, 0].astype(jnp.bfloat16), jnp.uint16).astype(jnp.uint32)
        b2 = jax.lax.bitcast_convert_type(
            sp[:, 1].astype(jnp.bfloat16), jnp.uint16).astype(jnp.uint32)
        s12 = jax.lax.bitcast_convert_type(b1 | (b2 << 16), jnp.int32)
        bk = jnp.full((16,), b[k], jnp.float32)
        partial = _sc_hop(h, s12, src, dst, bk)
        noise = SIGMA * jax.random.normal(
            jax.random.fold_in(jax.random.key(1), k), (N, D),
            dtype=jnp.float32)
        wnext = wpads[k + 1] if k + 1 < HOPS else wpad_zero
        h, sp = _tc_combine(partial, noise, wnext)
        outs.append(h)
    return jnp.stack(outs)


# speedup vs baseline: 1.0220x; 1.0198x over previous
import functools

import jax
import jax.numpy as jnp
from jax import lax
from jax.experimental import pallas as pl
from jax.experimental.pallas import tpu as pltpu
from jax.experimental.pallas import tpu_sc as plsc

N = 10000
E = 320000
D = 128
HOPS = 3
SIGMA = 0.1

NC = 2
NS = 16
NW = NC * NS
EPW = E // NW
KB = 80
NB = EPW // KB
RPT = 624
RTAIL = N - NS * RPT
ZR = 48
NZ = RPT // ZR

_SELU_SCALE = 1.0507009873554805
_SELU_ALPHA = 1.6732632423543772


def _sc_hop_body(h_hbm, s1_hbm, s2_hbm, src_hbm, dst_hbm, bk_hbm, out_hbm,
                 accum, s1v, s2v, rows0, rows1, srcb0, dstb0, srcb1, dstb1,
                 zbuf, bv, semi0, semi1, semr0, semr1):
    c = lax.axis_index("c")
    s = lax.axis_index("s")
    wid = c * NS + s
    ebase = wid * EPW
    rbase = s * RPT

    zero16 = jnp.zeros((16,), jnp.float32)

    def zrow(j, carry):
        for q in range(D // 16):
            zbuf[j, pl.ds(q * 16, 16)] = zero16
        return carry

    lax.fori_loop(0, ZR, zrow, 0)
    for z in range(NZ):
        pltpu.sync_copy(zbuf, accum.at[pl.ds(rbase + z * ZR, ZR)])

    @pl.when(s == 0)
    def _zero_tail():
        pltpu.sync_copy(zbuf.at[pl.ds(0, RTAIL)],
                        accum.at[pl.ds(NS * RPT, RTAIL)])

    plsc.subcore_barrier()

    pltpu.sync_copy(s1_hbm, s1v)
    pltpu.sync_copy(s2_hbm, s2v)
    pltpu.sync_copy(bk_hbm, bv)
    bvec = bv[...]

    bufs = ((srcb0, dstb0, rows0, semi0, semr0),
            (srcb1, dstb1, rows1, semi1, semr1))

    def idx_start(t, b):
        sb, db, _, semi, _ = bufs[b]
        eb = ebase + t * KB
        pltpu.make_async_copy(src_hbm.at[pl.ds(eb, KB)], sb, semi).start()
        pltpu.make_async_copy(dst_hbm.at[pl.ds(eb, KB)], db, semi).start()

    def idx_wait(b):
        sb, db, _, semi, _ = bufs[b]
        pltpu.make_async_copy(src_hbm.at[pl.ds(0, KB)], sb, semi).wait()
        pltpu.make_async_copy(dst_hbm.at[pl.ds(0, KB)], db, semi).wait()

    def gather_start(b):
        sb, _, rw, _, semr = bufs[b]
        pltpu.make_async_copy(h_hbm.at[sb], rw, semr).start()

    def gather_wait(b):
        sb, _, rw, _, semr = bufs[b]
        pltpu.make_async_copy(h_hbm.at[sb], rw, semr).wait()

    def compute_scatter(b):
        sb, db, rw, _, _ = bufs[b]
        for q in range(KB // 16):
            si = sb[pl.ds(q * 16, 16)]
            di = db[pl.ds(q * 16, 16)]
            e = (plsc.load_gather(s1v, [si]) + plsc.load_gather(s2v, [di])
                 + bvec)
            selu = _SELU_SCALE * jnp.where(
                e > 0, e, _SELU_ALPHA * (jnp.exp(e) - 1.0))
            avec = 1.0 / (1.0 + jnp.exp(-selu))
            for l in range(16):
                j = q * 16 + l
                av = avec[l]
                for qq in range(D // 16):
                    rw[j, pl.ds(qq * 16, 16)] = (
                        rw[j, pl.ds(qq * 16, 16)] * av)
        pltpu.sync_copy(rw, accum.at[db], add=True)

    idx_start(0, 0)
    idx_start(1, 1)
    idx_wait(0)
    gather_start(0)

    def bbody(i, carry):
        gather_wait(0)
        idx_wait(1)
        gather_start(1)
        compute_scatter(0)
        idx_start(2 * i + 2, 0)
        gather_wait(1)
        idx_wait(0)
        gather_start(0)
        compute_scatter(1)

        @pl.when(i < (NB - 1) // 2 - 1)
        def _more():
            idx_start(2 * i + 3, 1)

        return carry

    lax.fori_loop(0, (NB - 1) // 2, bbody, 0)
    gather_wait(0)
    compute_scatter(0)
    plsc.subcore_barrier()

    pltpu.sync_copy(accum.at[pl.ds(rbase, RPT)],
                    out_hbm.at[c, pl.ds(rbase, RPT)])

    @pl.when(s == 0)
    def _copy_tail():
        pltpu.sync_copy(accum.at[pl.ds(NS * RPT, RTAIL)],
                        out_hbm.at[c, pl.ds(NS * RPT, RTAIL)])


_sc_hop = functools.partial(
    pl.kernel,
    out_type=jax.ShapeDtypeStruct((NC, N, D), jnp.float32),
    mesh=plsc.VectorSubcoreMesh(core_axis_name="c", subcore_axis_name="s"),
    compiler_params=pltpu.CompilerParams(needs_layout_passes=False),
    scratch_types=[
        pltpu.VMEM_SHARED((N, D), jnp.float32),
        pltpu.VMEM((N,), jnp.float32),
        pltpu.VMEM((N,), jnp.float32),
        pltpu.VMEM((KB, D), jnp.float32),
        pltpu.VMEM((KB, D), jnp.float32),
        pltpu.VMEM((KB,), jnp.int32),
        pltpu.VMEM((KB,), jnp.int32),
        pltpu.VMEM((KB,), jnp.int32),
        pltpu.VMEM((KB,), jnp.int32),
        pltpu.VMEM((ZR, D), jnp.float32),
        pltpu.VMEM((16,), jnp.float32),
        pltpu.SemaphoreType.DMA,
        pltpu.SemaphoreType.DMA,
        pltpu.SemaphoreType.DMA,
        pltpu.SemaphoreType.DMA,
    ],
)(_sc_hop_body)


def _prep_body(x_ref, wp_ref, h_ref, sp_ref):
    xx = x_ref[...]
    nrm = jnp.sqrt(jnp.sum(xx * xx, axis=1, keepdims=True))
    h = xx / jnp.maximum(nrm, 1e-12)
    h_ref[...] = h
    sp_ref[...] = jnp.dot(h, wp_ref[...], preferred_element_type=jnp.float32)


def _tc_prep(x, wpad):
    return pl.pallas_call(
        _prep_body,
        out_shape=[jax.ShapeDtypeStruct((N, D), jnp.float32),
                   jax.ShapeDtypeStruct((N, 128), jnp.float32)],
    )(x, wpad)


def _combine_body(p_ref, nz_ref, wp_ref, h_ref, sp_ref):
    y = p_ref[0] + p_ref[1] + nz_ref[...]
    nrm = jnp.sqrt(jnp.sum(y * y, axis=1, keepdims=True))
    h = y / jnp.maximum(nrm, 1e-12)
    h_ref[...] = h
    sp_ref[...] = jnp.dot(h, wp_ref[...], preferred_element_type=jnp.float32)


def _tc_combine(partial, noise, wpad):
    return pl.pallas_call(
        _combine_body,
        out_shape=[jax.ShapeDtypeStruct((N, D), jnp.float32),
                   jax.ShapeDtypeStruct((N, 128), jnp.float32)],
    )(partial, noise, wpad)


def kernel(x, edge_index, W, b):
    src = edge_index[0]
    dst = edge_index[1]
    wpads = []
    for k in range(HOPS):
        wp = jnp.zeros((D, 128), jnp.float32)
        wp = wp.at[:, 0].set(W[k, :D]).at[:, 1].set(W[k, D:])
        wpads.append(wp)
    wpad_zero = jnp.zeros((D, 128), jnp.float32)

    h, sp = _tc_prep(x, wpads[0])
    outs = [h]
    for k in range(HOPS):
        s1 = sp[:, 0]
        s2 = sp[:, 1]
        bk = jnp.full((16,), b[k], jnp.float32)
        partial = _sc_hop(h, s1, s2, src, dst, bk)
        noise = SIGMA * jax.random.normal(
            jax.random.fold_in(jax.random.key(1), k), (N, D),
            dtype=jnp.float32)
        wnext = wpads[k + 1] if k + 1 < HOPS else wpad_zero
        h, sp = _tc_combine(partial, noise, wnext)
        outs.append(h)
    return jnp.stack(outs)
